# baseline (speedup 1.0000x reference)
import functools

import jax
import jax.numpy as jnp
from jax import lax
from jax.experimental import pallas as pl
from jax.experimental.pallas import tpu as pltpu
from jax.experimental.pallas import tpu_sc as plsc

N = 10000
E = 320000
D_IN = 128
H1, F1 = 8, 8
H2, F2 = 1, 40

NP = 10240
TILES = 32
CHUNK = 128
ACCN = 10048
RPT = ACCN // 16
RFULL = RPT // CHUNK
RTAIL = RPT - RFULL * CHUNK
ABLK = 1256
NCHUNK = (((E + TILES * CHUNK - 1) // (TILES * CHUNK)) + 3) // 4 * 4
EPT = NCHUNK * CHUNK
EP = EPT * TILES
BLK = 1024


def _make_sc_edge_kernel(width, n_heads):
  accw = width + 16
  nq = width // 16
  mesh = plsc.VectorSubcoreMesh(core_axis_name="c", subcore_axis_name="s",
                                num_cores=2, num_subcores=16)

  @functools.partial(
      pl.kernel,
      out_type=jax.ShapeDtypeStruct((2, ACCN, accw), jnp.float32),
      mesh=mesh,
      compiler_params=pltpu.CompilerParams(use_tc_tiling_on_sc=False),
      scratch_types=[
          pltpu.VMEM((CHUNK,), jnp.int32),
          pltpu.VMEM((CHUNK,), jnp.int32),
          pltpu.VMEM((CHUNK,), jnp.int32),
          pltpu.VMEM((CHUNK,), jnp.int32),
          pltpu.VMEM((CHUNK,), jnp.int32),
          pltpu.VMEM((CHUNK,), jnp.int32),
          pltpu.VMEM((CHUNK,), jnp.int32),
          pltpu.VMEM((CHUNK,), jnp.int32),
          pltpu.VMEM((CHUNK, width), jnp.float32),
          pltpu.VMEM((CHUNK, width), jnp.float32),
          pltpu.VMEM((CHUNK, width), jnp.float32),
          pltpu.VMEM((CHUNK, width), jnp.float32),
          pltpu.VMEM((CHUNK, accw), jnp.float32),
          pltpu.VMEM((CHUNK, accw), jnp.float32),
          pltpu.VMEM((width,), jnp.float32),
          pltpu.VMEM((4, 16), jnp.float32),
          pltpu.VMEM((16,), jnp.int32),
          pltpu.VMEM_SHARED((ACCN, accw), jnp.float32),
          pltpu.SemaphoreType.DMA,
          pltpu.SemaphoreType.DMA,
          pltpu.SemaphoreType.DMA,
          pltpu.SemaphoreType.DMA,
          pltpu.SemaphoreType.DMA,
          pltpu.SemaphoreType.DMA,
          pltpu.SemaphoreType.DMA,
          pltpu.SemaphoreType.DMA,
      ],
  )
  def edge_kernel(src_hbm, dst_hbm, fs_hbm, fd_hbm, attn_hbm,
                  cmask_hbm, didx_hbm, out_hbm,
                  idx_s0, idx_s1, idx_s2, idx_s3,
                  idx_d0, idx_d1, idx_d2, idx_d3,
                  rows_s0, rows_s1, rows_d0, rows_d1,
                  contrib0, contrib1, attn_v, cmask_v, didx_v, acc,
                  ii0, ii1, ii2, ii3, gg0, gg1, ss0, ss1):
    cid = lax.axis_index("c")
    sid = lax.axis_index("s")
    wid = sid * 2 + cid
    tile_base = wid * EPT

    idx_s = [idx_s0, idx_s1, idx_s2, idx_s3]
    idx_d = [idx_d0, idx_d1, idx_d2, idx_d3]
    rows_s = [rows_s0, rows_s1]
    rows_d = [rows_d0, rows_d1]
    contrib = [contrib0, contrib1]
    ii = [ii0, ii1, ii2, ii3]
    gg = [gg0, gg1]
    ss = [ss0, ss1]

    zeros16 = jnp.zeros((16,), jnp.float32)

    def zero_row(i, _):
      def zero_col(j, _):
        contrib0[i, pl.ds(j * 16, 16)] = zeros16
        return 0
      return lax.fori_loop(0, accw // 16, zero_col, 0)
    lax.fori_loop(0, CHUNK, zero_row, 0)
    for r in range(RFULL):
      pltpu.sync_copy(contrib0,
                      acc.at[pl.ds(sid * RPT + r * CHUNK, CHUNK)])
    pltpu.sync_copy(contrib0.at[pl.ds(0, RTAIL)],
                    acc.at[pl.ds(sid * RPT + RFULL * CHUNK, RTAIL)])
    plsc.subcore_barrier()

    pltpu.sync_copy(attn_hbm, attn_v)
    attn_q = [attn_v[pl.ds(q * 16, 16)] for q in range(nq)]
    if n_heads > 1:
      pltpu.sync_copy(cmask_hbm, cmask_v)
      pltpu.sync_copy(didx_hbm, didx_v)

    lanes = lax.iota(jnp.int32, 16)
    perms = [lanes ^ 1, lanes ^ 2, lanes ^ 4, lanes ^ 8]

    def bfly(v, p):
      return v + jnp.take_along_axis(v, p, axis=0,
                                     mode="promise_in_bounds")

    def chunk_base(c):
      return tile_base + jnp.minimum(c, NCHUNK - 1) * CHUNK

    def issue_idx(c, j):
      base = chunk_base(c)
      pltpu.async_copy(src_hbm.at[pl.ds(base, CHUNK)], idx_s[j], ii[j])
      pltpu.async_copy(dst_hbm.at[pl.ds(base, CHUNK)], idx_d[j], ii[j])

    def wait_idx(c, j):
      base = chunk_base(c)
      pltpu.make_async_copy(src_hbm.at[pl.ds(base, CHUNK)], idx_s[j],
                            ii[j]).wait()
      pltpu.make_async_copy(dst_hbm.at[pl.ds(base, CHUNK)], idx_d[j],
                            ii[j]).wait()

    def issue_gather(j, p):
      pltpu.async_copy(fs_hbm.at[idx_s[j]], rows_s[p], gg[p])
      pltpu.async_copy(fd_hbm.at[idx_d[j]], rows_d[p], gg[p])

    def wait_gather(j, p):
      pltpu.make_async_copy(fs_hbm.at[idx_s[j]], rows_s[p], gg[p]).wait()
      pltpu.make_async_copy(fd_hbm.at[idx_d[j]], rows_d[p], gg[p]).wait()

    def issue_scatter(j, p):
      pltpu.async_copy(contrib[p], acc.at[idx_d[j]], ss[p], add=True)

    def wait_scatter(j, p):
      pltpu.make_async_copy(contrib[p], acc.at[idx_d[j]], ss[p]).wait()

    if n_heads > 1:
      cvec = [cmask_v[q, pl.ds(0, 16)] for q in range(nq)]
      didx = didx_v[pl.ds(0, 16)]

      def make_edge_body(p):
        def edge_body(e, _):
          den_pre = zeros16
          for q in range(nq):
            s_q = rows_s[p][e, pl.ds(q * 16, 16)]
            d_q = rows_d[p][e, pl.ds(q * 16, 16)]
            t = s_q + d_q
            lr = jnp.maximum(t, 0.2 * t)
            m = lr * attn_q[q]
            for pidx in perms[:3]:
              m = bfly(m, pidx)
            pv = jnp.exp(m)
            contrib[p][e, pl.ds(q * 16, 16)] = pv * s_q
            den_pre = den_pre + pv * cvec[q]
          den = jnp.take_along_axis(den_pre, didx, axis=0,
                                    mode="promise_in_bounds")
          contrib[p][e, pl.ds(width, 16)] = den
          return 0
        return edge_body
    else:
      def make_edge_body(p):
        def edge_body(e, _):
          s_qs = []
          tot = zeros16
          for q in range(nq):
            s_q = rows_s[p][e, pl.ds(q * 16, 16)]
            d_q = rows_d[p][e, pl.ds(q * 16, 16)]
            t = s_q + d_q
            lr = jnp.maximum(t, 0.2 * t)
            tot = tot + lr * attn_q[q]
            s_qs.append(s_q)
          for p_idx in perms:
            tot = bfly(tot, p_idx)
          pv = jnp.exp(tot)
          for q in range(nq):
            contrib[p][e, pl.ds(q * 16, 16)] = pv * s_qs[q]
          contrib[p][e, pl.ds(width, 16)] = pv
          return 0
        return edge_body

    edge_bodies = [make_edge_body(0), make_edge_body(1)]

    def sub_iter(c, k, steady):
      p = k % 2
      q = 1 - p
      kn = (k + 1) % 4
      kf = (k + 2) % 4
      wait_idx(c + 1, kn)
      issue_gather(kn, q)
      wait_gather(k, p)
      if steady:
        wait_scatter(kf, p)
      lax.fori_loop(0, CHUNK, edge_bodies[p], 0, unroll=8)
      issue_scatter(k, p)
      issue_idx(c + 2, kf)

    issue_idx(0, 0)
    wait_idx(0, 0)
    issue_gather(0, 0)
    issue_idx(1, 1)

    sub_iter(0, 0, False)
    sub_iter(1, 1, False)
    sub_iter(2, 2, True)
    sub_iter(3, 3, True)

    def group_body(g, _):
      c0 = 4 * g
      sub_iter(c0, 0, True)
      sub_iter(c0 + 1, 1, True)
      sub_iter(c0 + 2, 2, True)
      sub_iter(c0 + 3, 3, True)
      return 0
    lax.fori_loop(1, NCHUNK // 4, group_body, 0)

    wait_gather(0, 0)
    wait_idx(NCHUNK + 1, 1)
    wait_scatter(2, 0)
    wait_scatter(3, 1)

    plsc.subcore_barrier()
    for r in range(RFULL):
      off = sid * RPT + r * CHUNK
      pltpu.sync_copy(acc.at[pl.ds(off, CHUNK)],
                      out_hbm.at[cid, pl.ds(off, CHUNK)])
    off = sid * RPT + RFULL * CHUNK
    pltpu.sync_copy(acc.at[pl.ds(off, RTAIL)],
                    out_hbm.at[cid, pl.ds(off, RTAIL)])

  return edge_kernel


def _mm1_body(x_ref, ws_ref, wd_ref, fs_ref, fd_ref):
  xb = x_ref[...]
  fs_ref[...] = jnp.dot(xb, ws_ref[...], preferred_element_type=jnp.float32)
  fd_ref[...] = jnp.dot(xb, wd_ref[...], preferred_element_type=jnp.float32)


def _mid_body(acc_ref, b1_ref, ws_ref, wd_ref, fs2_ref, fd2_ref):
  a = acc_ref[...]
  s = a[0] + a[1]
  num = s[:, :64]
  den = jnp.repeat(s[:, 64:72], 8, axis=1)
  nz = den != 0.0
  h = jnp.where(nz, num / jnp.where(nz, den, 1.0), 0.0) + b1_ref[...]
  h = jnp.where(h > 0.0, h, jnp.exp(h) - 1.0)
  fs2_ref[...] = jnp.dot(h, ws_ref[...], preferred_element_type=jnp.float32)
  fd2_ref[...] = jnp.dot(h, wd_ref[...], preferred_element_type=jnp.float32)


def _fin_body(acc_ref, b2_ref, o_ref):
  a = acc_ref[...]
  s = a[0] + a[1]
  num = s[:, :48]
  den = s[:, 48:49]
  nz = den != 0.0
  o_ref[...] = jnp.where(nz, num / jnp.where(nz, den, 1.0), 0.0) + b2_ref[...]


def kernel(x, edge_index, W1_src, W1_dst, attn1, b1, W2_src, W2_dst,
           attn2, b2):
  f32 = jnp.float32
  npad = NP - N
  x_p = jnp.pad(x, ((0, npad), (0, 0)))
  pad_e = jnp.full((EP - E,), N, jnp.int32)
  src = jnp.concatenate([edge_index[0], pad_e])
  dst = jnp.concatenate([edge_index[1], pad_e])

  cmask_l = [[0.0] * 16 for _ in range(4)]
  for q in range(4):
    cmask_l[q][2 * q] = 1.0
    cmask_l[q][2 * q + 9] = 1.0
  cmask = jnp.array(cmask_l, f32)
  didx = jnp.array([l + 8 if (l % 2 == 1 and l < 8) else l
                    for l in range(16)], jnp.int32)

  attn1_flat = attn1.reshape(H1 * F1).astype(f32)
  attn2_flat = jnp.pad(attn2.reshape(H2 * F2), (0, 8)).astype(f32)
  W2s_p = jnp.pad(W2_src, ((0, 0), (0, 8)))
  W2d_p = jnp.pad(W2_dst, ((0, 0), (0, 8)))
  b1_2d = b1.reshape(1, 64)
  b2_2d = jnp.pad(b2, (0, 8)).reshape(1, 48)

  grid = (NP // BLK,)
  fs1, fd1 = pl.pallas_call(
      _mm1_body,
      grid=grid,
      in_specs=[
          pl.BlockSpec((BLK, D_IN), lambda i: (i, 0)),
          pl.BlockSpec((D_IN, 64), lambda i: (0, 0)),
          pl.BlockSpec((D_IN, 64), lambda i: (0, 0)),
      ],
      out_specs=[
          pl.BlockSpec((BLK, 64), lambda i: (i, 0)),
          pl.BlockSpec((BLK, 64), lambda i: (i, 0)),
      ],
      out_shape=[jax.ShapeDtypeStruct((NP, 64), f32)] * 2,
  )(x_p, W1_src, W1_dst)

  edge1 = _make_sc_edge_kernel(64, H1)
  acc1 = edge1(src, dst, fs1, fd1, attn1_flat, cmask, didx)

  agrid = (ACCN // ABLK,)
  fs2, fd2 = pl.pallas_call(
      _mid_body,
      grid=agrid,
      in_specs=[
          pl.BlockSpec((2, ABLK, 80), lambda i: (0, i, 0)),
          pl.BlockSpec((1, 64), lambda i: (0, 0)),
          pl.BlockSpec((64, 48), lambda i: (0, 0)),
          pl.BlockSpec((64, 48), lambda i: (0, 0)),
      ],
      out_specs=[
          pl.BlockSpec((ABLK, 48), lambda i: (i, 0)),
          pl.BlockSpec((ABLK, 48), lambda i: (i, 0)),
      ],
      out_shape=[jax.ShapeDtypeStruct((ACCN, 48), f32)] * 2,
  )(acc1, b1_2d, W2s_p, W2d_p)

  edge2 = _make_sc_edge_kernel(48, H2)
  acc2 = edge2(src, dst, fs2, fd2, attn2_flat, cmask, didx)

  out = pl.pallas_call(
      _fin_body,
      grid=agrid,
      in_specs=[
          pl.BlockSpec((2, ABLK, 64), lambda i: (0, i, 0)),
          pl.BlockSpec((1, 48), lambda i: (0, 0)),
      ],
      out_specs=pl.BlockSpec((ABLK, 48), lambda i: (i, 0)),
      out_shape=jax.ShapeDtypeStruct((ACCN, 48), f32),
  )(acc2, b2_2d)

  return out[:N, :H2 * F2]


# speedup vs baseline: 1.4314x; 1.4314x over previous
import functools

import jax
import jax.numpy as jnp
from jax import lax
from jax.experimental import pallas as pl
from jax.experimental.pallas import tpu as pltpu
from jax.experimental.pallas import tpu_sc as plsc

N = 10000
E = 320000
D_IN = 128
H1, F1 = 8, 8
H2, F2 = 1, 40

NP = 10240
TILES = 32
CHUNK = 128
ACCN = 10048
RPT = ACCN // 16
RFULL = RPT // CHUNK
RTAIL = RPT - RFULL * CHUNK
ABLK = 1256
NCHUNK = (((E + TILES * CHUNK - 1) // (TILES * CHUNK)) + 3) // 4 * 4
EPT = NCHUNK * CHUNK
EP = EPT * TILES
BLK = 1024


def _make_sc_edge_kernel(width, n_heads):
  accw = width + 16
  nq = width // 16
  mesh = plsc.VectorSubcoreMesh(core_axis_name="c", subcore_axis_name="s",
                                num_cores=2, num_subcores=16)

  @functools.partial(
      pl.kernel,
      out_type=jax.ShapeDtypeStruct((2, ACCN, accw), jnp.float32),
      mesh=mesh,
      compiler_params=pltpu.CompilerParams(use_tc_tiling_on_sc=False),
      scratch_types=[
          pltpu.VMEM((CHUNK,), jnp.int32),
          pltpu.VMEM((CHUNK,), jnp.int32),
          pltpu.VMEM((CHUNK,), jnp.int32),
          pltpu.VMEM((CHUNK,), jnp.int32),
          pltpu.VMEM((CHUNK,), jnp.int32),
          pltpu.VMEM((CHUNK,), jnp.int32),
          pltpu.VMEM((CHUNK,), jnp.int32),
          pltpu.VMEM((CHUNK,), jnp.int32),
          pltpu.VMEM((CHUNK, width), jnp.float32),
          pltpu.VMEM((CHUNK, width), jnp.float32),
          pltpu.VMEM((CHUNK, width), jnp.float32),
          pltpu.VMEM((CHUNK, width), jnp.float32),
          pltpu.VMEM((CHUNK, accw), jnp.float32),
          pltpu.VMEM((CHUNK, accw), jnp.float32),
          pltpu.VMEM((width,), jnp.float32),
          pltpu.VMEM((4, 16), jnp.float32),
          pltpu.VMEM((16,), jnp.int32),
          pltpu.VMEM_SHARED((ACCN, accw), jnp.float32),
          pltpu.SemaphoreType.DMA,
          pltpu.SemaphoreType.DMA,
          pltpu.SemaphoreType.DMA,
          pltpu.SemaphoreType.DMA,
          pltpu.SemaphoreType.DMA,
          pltpu.SemaphoreType.DMA,
          pltpu.SemaphoreType.DMA,
          pltpu.SemaphoreType.DMA,
      ],
  )
  def edge_kernel(src_hbm, dst_hbm, fs_hbm, fd_hbm, attn_hbm,
                  cmask_hbm, didx_hbm, out_hbm,
                  idx_s0, idx_s1, idx_s2, idx_s3,
                  idx_d0, idx_d1, idx_d2, idx_d3,
                  rows_s0, rows_s1, rows_d0, rows_d1,
                  contrib0, contrib1, attn_v, cmask_v, didx_v, acc,
                  ii0, ii1, ii2, ii3, gg0, gg1, ss0, ss1):
    cid = lax.axis_index("c")
    sid = lax.axis_index("s")
    wid = sid * 2 + cid
    tile_base = wid * EPT

    idx_s = [idx_s0, idx_s1, idx_s2, idx_s3]
    idx_d = [idx_d0, idx_d1, idx_d2, idx_d3]
    rows_s = [rows_s0, rows_s1]
    rows_d = [rows_d0, rows_d1]
    contrib = [contrib0, contrib1]
    ii = [ii0, ii1, ii2, ii3]
    gg = [gg0, gg1]
    ss = [ss0, ss1]

    zeros16 = jnp.zeros((16,), jnp.float32)

    def zero_row(i, _):
      def zero_col(j, _):
        contrib0[i, pl.ds(j * 16, 16)] = zeros16
        return 0
      return lax.fori_loop(0, accw // 16, zero_col, 0)
    lax.fori_loop(0, CHUNK, zero_row, 0)
    for r in range(RFULL):
      pltpu.sync_copy(contrib0,
                      acc.at[pl.ds(sid * RPT + r * CHUNK, CHUNK)])
    pltpu.sync_copy(contrib0.at[pl.ds(0, RTAIL)],
                    acc.at[pl.ds(sid * RPT + RFULL * CHUNK, RTAIL)])
    plsc.subcore_barrier()

    pltpu.sync_copy(attn_hbm, attn_v)
    attn_q = [attn_v[pl.ds(q * 16, 16)] for q in range(nq)]
    if n_heads > 1:
      pltpu.sync_copy(cmask_hbm, cmask_v)
      pltpu.sync_copy(didx_hbm, didx_v)

    lanes = lax.iota(jnp.int32, 16)
    perms = [lanes ^ 1, lanes ^ 2, lanes ^ 4, lanes ^ 8]

    def bfly(v, p):
      return v + jnp.take_along_axis(v, p, axis=0,
                                     mode="promise_in_bounds")

    def chunk_base(c):
      return tile_base + jnp.minimum(c, NCHUNK - 1) * CHUNK

    def issue_idx(c, j):
      base = chunk_base(c)
      pltpu.async_copy(src_hbm.at[pl.ds(base, CHUNK)], idx_s[j], ii[j])
      pltpu.async_copy(dst_hbm.at[pl.ds(base, CHUNK)], idx_d[j], ii[j])

    def wait_idx(c, j):
      base = chunk_base(c)
      pltpu.make_async_copy(src_hbm.at[pl.ds(base, CHUNK)], idx_s[j],
                            ii[j]).wait()
      pltpu.make_async_copy(dst_hbm.at[pl.ds(base, CHUNK)], idx_d[j],
                            ii[j]).wait()

    def issue_gather(j, p):
      pltpu.async_copy(fs_hbm.at[idx_s[j]], rows_s[p], gg[p])
      pltpu.async_copy(fd_hbm.at[idx_d[j]], rows_d[p], gg[p])

    def wait_gather(j, p):
      pltpu.make_async_copy(fs_hbm.at[idx_s[j]], rows_s[p], gg[p]).wait()
      pltpu.make_async_copy(fd_hbm.at[idx_d[j]], rows_d[p], gg[p]).wait()

    def issue_scatter(j, p):
      pltpu.async_copy(contrib[p], acc.at[idx_d[j]], ss[p], add=True)

    def wait_scatter(j, p):
      pltpu.make_async_copy(contrib[p], acc.at[idx_d[j]], ss[p]).wait()

    if n_heads > 1:
      cvec = [cmask_v[q, pl.ds(0, 16)] for q in range(nq)]
      didx = didx_v[pl.ds(0, 16)]

      def make_edge_body(p):
        def edge_body(i, _):
          e0 = 2 * i
          e1 = e0 + 1
          den_a = zeros16
          den_b = zeros16
          for q in range(nq):
            sa = rows_s[p][e0, pl.ds(q * 16, 16)]
            sb = rows_s[p][e1, pl.ds(q * 16, 16)]
            da = rows_d[p][e0, pl.ds(q * 16, 16)]
            db = rows_d[p][e1, pl.ds(q * 16, 16)]
            ta = sa + da
            tb = sb + db
            la = jnp.maximum(ta, 0.2 * ta)
            lb = jnp.maximum(tb, 0.2 * tb)
            ma = la * attn_q[q]
            mb = lb * attn_q[q]
            for pidx in perms[:3]:
              ma = bfly(ma, pidx)
              mb = bfly(mb, pidx)
            pa = jnp.exp(ma)
            pb = jnp.exp(mb)
            contrib[p][e0, pl.ds(q * 16, 16)] = pa * sa
            contrib[p][e1, pl.ds(q * 16, 16)] = pb * sb
            den_a = den_a + pa * cvec[q]
            den_b = den_b + pb * cvec[q]
          contrib[p][e0, pl.ds(width, 16)] = jnp.take_along_axis(
              den_a, didx, axis=0, mode="promise_in_bounds")
          contrib[p][e1, pl.ds(width, 16)] = jnp.take_along_axis(
              den_b, didx, axis=0, mode="promise_in_bounds")
          return 0
        return edge_body
      eloop_n, eloop_unroll = CHUNK // 2, 4
    else:
      def make_edge_body(p):
        def edge_body(e, _):
          s_qs = []
          tot = zeros16
          for q in range(nq):
            s_q = rows_s[p][e, pl.ds(q * 16, 16)]
            d_q = rows_d[p][e, pl.ds(q * 16, 16)]
            t = s_q + d_q
            lr = jnp.maximum(t, 0.2 * t)
            tot = tot + lr * attn_q[q]
            s_qs.append(s_q)
          for p_idx in perms:
            tot = bfly(tot, p_idx)
          pv = jnp.exp(tot)
          for q in range(nq):
            contrib[p][e, pl.ds(q * 16, 16)] = pv * s_qs[q]
          contrib[p][e, pl.ds(width, 16)] = pv
          return 0
        return edge_body
      eloop_n, eloop_unroll = CHUNK, 8

    edge_bodies = [make_edge_body(0), make_edge_body(1)]

    def sub_iter(c, k, steady):
      p = k % 2
      q = 1 - p
      kn = (k + 1) % 4
      kf = (k + 2) % 4
      wait_idx(c + 1, kn)
      issue_gather(kn, q)
      wait_gather(k, p)
      if steady:
        wait_scatter(kf, p)
      lax.fori_loop(0, eloop_n, edge_bodies[p], 0, unroll=eloop_unroll)
      issue_scatter(k, p)
      issue_idx(c + 2, kf)

    issue_idx(0, 0)
    wait_idx(0, 0)
    issue_gather(0, 0)
    issue_idx(1, 1)

    sub_iter(0, 0, False)
    sub_iter(1, 1, False)
    sub_iter(2, 2, True)
    sub_iter(3, 3, True)

    def group_body(g, _):
      c0 = 4 * g
      sub_iter(c0, 0, True)
      sub_iter(c0 + 1, 1, True)
      sub_iter(c0 + 2, 2, True)
      sub_iter(c0 + 3, 3, True)
      return 0
    lax.fori_loop(1, NCHUNK // 4, group_body, 0)

    wait_gather(0, 0)
    wait_idx(NCHUNK + 1, 1)
    wait_scatter(2, 0)
    wait_scatter(3, 1)

    plsc.subcore_barrier()
    for r in range(RFULL):
      off = sid * RPT + r * CHUNK
      pltpu.sync_copy(acc.at[pl.ds(off, CHUNK)],
                      out_hbm.at[cid, pl.ds(off, CHUNK)])
    off = sid * RPT + RFULL * CHUNK
    pltpu.sync_copy(acc.at[pl.ds(off, RTAIL)],
                    out_hbm.at[cid, pl.ds(off, RTAIL)])

  return edge_kernel


def _mm1_body(x_ref, ws_ref, wd_ref, fs_ref, fd_ref):
  xb = x_ref[...]
  fs_ref[...] = jnp.dot(xb, ws_ref[...], preferred_element_type=jnp.float32)
  fd_ref[...] = jnp.dot(xb, wd_ref[...], preferred_element_type=jnp.float32)


def _mid_body(acc_ref, b1_ref, ws_ref, wd_ref, fs2_ref, fd2_ref):
  a = acc_ref[...]
  s = a[0] + a[1]
  num = s[:, :64]
  den = jnp.repeat(s[:, 64:72], 8, axis=1)
  nz = den != 0.0
  h = jnp.where(nz, num / jnp.where(nz, den, 1.0), 0.0) + b1_ref[...]
  h = jnp.where(h > 0.0, h, jnp.exp(h) - 1.0)
  fs2_ref[...] = jnp.dot(h, ws_ref[...], preferred_element_type=jnp.float32)
  fd2_ref[...] = jnp.dot(h, wd_ref[...], preferred_element_type=jnp.float32)


def _fin_body(acc_ref, b2_ref, o_ref):
  a = acc_ref[...]
  s = a[0] + a[1]
  num = s[:, :48]
  den = s[:, 48:49]
  nz = den != 0.0
  o_ref[...] = jnp.where(nz, num / jnp.where(nz, den, 1.0), 0.0) + b2_ref[...]


def kernel(x, edge_index, W1_src, W1_dst, attn1, b1, W2_src, W2_dst,
           attn2, b2):
  f32 = jnp.float32
  npad = NP - N
  x_p = jnp.pad(x, ((0, npad), (0, 0)))
  pad_e = jnp.full((EP - E,), N, jnp.int32)
  src = jnp.concatenate([edge_index[0], pad_e])
  dst = jnp.concatenate([edge_index[1], pad_e])

  cmask_l = [[0.0] * 16 for _ in range(4)]
  for q in range(4):
    cmask_l[q][2 * q] = 1.0
    cmask_l[q][2 * q + 9] = 1.0
  cmask = jnp.array(cmask_l, f32)
  didx = jnp.array([l + 8 if (l % 2 == 1 and l < 8) else l
                    for l in range(16)], jnp.int32)

  attn1_flat = attn1.reshape(H1 * F1).astype(f32)
  attn2_flat = jnp.pad(attn2.reshape(H2 * F2), (0, 8)).astype(f32)
  W2s_p = jnp.pad(W2_src, ((0, 0), (0, 8)))
  W2d_p = jnp.pad(W2_dst, ((0, 0), (0, 8)))
  b1_2d = b1.reshape(1, 64)
  b2_2d = jnp.pad(b2, (0, 8)).reshape(1, 48)

  grid = (NP // BLK,)
  fs1, fd1 = pl.pallas_call(
      _mm1_body,
      grid=grid,
      in_specs=[
          pl.BlockSpec((BLK, D_IN), lambda i: (i, 0)),
          pl.BlockSpec((D_IN, 64), lambda i: (0, 0)),
          pl.BlockSpec((D_IN, 64), lambda i: (0, 0)),
      ],
      out_specs=[
          pl.BlockSpec((BLK, 64), lambda i: (i, 0)),
          pl.BlockSpec((BLK, 64), lambda i: (i, 0)),
      ],
      out_shape=[jax.ShapeDtypeStruct((NP, 64), f32)] * 2,
  )(x_p, W1_src, W1_dst)

  edge1 = _make_sc_edge_kernel(64, H1)
  acc1 = edge1(src, dst, fs1, fd1, attn1_flat, cmask, didx)

  agrid = (ACCN // ABLK,)
  fs2, fd2 = pl.pallas_call(
      _mid_body,
      grid=agrid,
      in_specs=[
          pl.BlockSpec((2, ABLK, 80), lambda i: (0, i, 0)),
          pl.BlockSpec((1, 64), lambda i: (0, 0)),
          pl.BlockSpec((64, 48), lambda i: (0, 0)),
          pl.BlockSpec((64, 48), lambda i: (0, 0)),
      ],
      out_specs=[
          pl.BlockSpec((ABLK, 48), lambda i: (i, 0)),
          pl.BlockSpec((ABLK, 48), lambda i: (i, 0)),
      ],
      out_shape=[jax.ShapeDtypeStruct((ACCN, 48), f32)] * 2,
  )(acc1, b1_2d, W2s_p, W2d_p)

  edge2 = _make_sc_edge_kernel(48, H2)
  acc2 = edge2(src, dst, fs2, fd2, attn2_flat, cmask, didx)

  out = pl.pallas_call(
      _fin_body,
      grid=agrid,
      in_specs=[
          pl.BlockSpec((2, ABLK, 64), lambda i: (0, i, 0)),
          pl.BlockSpec((1, 48), lambda i: (0, 0)),
      ],
      out_specs=pl.BlockSpec((ABLK, 48), lambda i: (i, 0)),
      out_shape=jax.ShapeDtypeStruct((ACCN, 48), f32),
  )(acc2, b2_2d)

  return out[:N, :H2 * F2]


# speedup vs baseline: 1.5615x; 1.0909x over previous
import functools

import jax
import jax.numpy as jnp
from jax import lax
from jax.experimental import pallas as pl
from jax.experimental.pallas import tpu as pltpu
from jax.experimental.pallas import tpu_sc as plsc

N = 10000
E = 320000
D_IN = 128
H1, F1 = 8, 8
H2, F2 = 1, 40

NP = 10240
TILES = 32
CHUNK = 128
ACCN = 10048
RPT = ACCN // 16
RFULL = RPT // CHUNK
RTAIL = RPT - RFULL * CHUNK
ABLK = 1256
NCHUNK = (((E + TILES * CHUNK - 1) // (TILES * CHUNK)) + 3) // 4 * 4
EPT = NCHUNK * CHUNK
EP = EPT * TILES
BLK = 1024


def _make_sc_edge_kernel(width, n_heads):
  accw = width + 16
  nq = width // 16
  mesh = plsc.VectorSubcoreMesh(core_axis_name="c", subcore_axis_name="s",
                                num_cores=2, num_subcores=16)

  @functools.partial(
      pl.kernel,
      out_type=jax.ShapeDtypeStruct((2, ACCN, accw), jnp.float32),
      mesh=mesh,
      compiler_params=pltpu.CompilerParams(use_tc_tiling_on_sc=False),
      scratch_types=[
          pltpu.VMEM((CHUNK,), jnp.int32),
          pltpu.VMEM((CHUNK,), jnp.int32),
          pltpu.VMEM((CHUNK,), jnp.int32),
          pltpu.VMEM((CHUNK,), jnp.int32),
          pltpu.VMEM((CHUNK,), jnp.int32),
          pltpu.VMEM((CHUNK,), jnp.int32),
          pltpu.VMEM((CHUNK,), jnp.int32),
          pltpu.VMEM((CHUNK,), jnp.int32),
          pltpu.VMEM((CHUNK, width), jnp.float32),
          pltpu.VMEM((CHUNK, width), jnp.float32),
          pltpu.VMEM((CHUNK, width), jnp.float32),
          pltpu.VMEM((CHUNK, width), jnp.float32),
          pltpu.VMEM((CHUNK, accw), jnp.float32),
          pltpu.VMEM((CHUNK, accw), jnp.float32),
          pltpu.VMEM((width,), jnp.float32),
          pltpu.VMEM((4, 16), jnp.float32),
          pltpu.VMEM((16,), jnp.int32),
          pltpu.VMEM_SHARED((ACCN, accw), jnp.float32),
          pltpu.SemaphoreType.DMA,
          pltpu.SemaphoreType.DMA,
          pltpu.SemaphoreType.DMA,
          pltpu.SemaphoreType.DMA,
          pltpu.SemaphoreType.DMA,
          pltpu.SemaphoreType.DMA,
          pltpu.SemaphoreType.DMA,
          pltpu.SemaphoreType.DMA,
      ],
  )
  def edge_kernel(src_hbm, dst_hbm, fs_hbm, fd_hbm, attn_hbm,
                  cmask_hbm, didx_hbm, out_hbm,
                  idx_s0, idx_s1, idx_s2, idx_s3,
                  idx_d0, idx_d1, idx_d2, idx_d3,
                  rows_s0, rows_s1, rows_d0, rows_d1,
                  contrib0, contrib1, attn_v, cmask_v, didx_v, acc,
                  ii0, ii1, ii2, ii3, gg0, gg1, ss0, ss1):
    cid = lax.axis_index("c")
    sid = lax.axis_index("s")
    wid = sid * 2 + cid
    tile_base = wid * EPT

    idx_s = [idx_s0, idx_s1, idx_s2, idx_s3]
    idx_d = [idx_d0, idx_d1, idx_d2, idx_d3]
    rows_s = [rows_s0, rows_s1]
    rows_d = [rows_d0, rows_d1]
    contrib = [contrib0, contrib1]
    ii = [ii0, ii1, ii2, ii3]
    gg = [gg0, gg1]
    ss = [ss0, ss1]

    zeros16 = jnp.zeros((16,), jnp.float32)

    def zero_row(i, _):
      def zero_col(j, _):
        contrib0[i, pl.ds(j * 16, 16)] = zeros16
        return 0
      return lax.fori_loop(0, accw // 16, zero_col, 0)
    lax.fori_loop(0, CHUNK, zero_row, 0)
    for r in range(RFULL):
      pltpu.sync_copy(contrib0,
                      acc.at[pl.ds(sid * RPT + r * CHUNK, CHUNK)])
    pltpu.sync_copy(contrib0.at[pl.ds(0, RTAIL)],
                    acc.at[pl.ds(sid * RPT + RFULL * CHUNK, RTAIL)])
    plsc.subcore_barrier()

    pltpu.sync_copy(attn_hbm, attn_v)
    attn_q = [attn_v[pl.ds(q * 16, 16)] for q in range(nq)]
    if n_heads > 1:
      pltpu.sync_copy(cmask_hbm, cmask_v)
      pltpu.sync_copy(didx_hbm, didx_v)

    lanes = lax.iota(jnp.int32, 16)
    perms = [lanes ^ 1, lanes ^ 2, lanes ^ 4, lanes ^ 8]

    def bfly(v, p):
      return v + jnp.take_along_axis(v, p, axis=0,
                                     mode="promise_in_bounds")

    def chunk_base(c):
      return tile_base + jnp.minimum(c, NCHUNK - 1) * CHUNK

    def issue_idx(c, j):
      base = chunk_base(c)
      pltpu.async_copy(src_hbm.at[pl.ds(base, CHUNK)], idx_s[j], ii[j])
      pltpu.async_copy(dst_hbm.at[pl.ds(base, CHUNK)], idx_d[j], ii[j])

    def wait_idx(c, j):
      base = chunk_base(c)
      pltpu.make_async_copy(src_hbm.at[pl.ds(base, CHUNK)], idx_s[j],
                            ii[j]).wait()
      pltpu.make_async_copy(dst_hbm.at[pl.ds(base, CHUNK)], idx_d[j],
                            ii[j]).wait()

    def issue_gather(j, p):
      pltpu.async_copy(fs_hbm.at[idx_s[j]], rows_s[p], gg[p])
      pltpu.async_copy(fd_hbm.at[idx_d[j]], rows_d[p], gg[p])

    def wait_gather(j, p):
      pltpu.make_async_copy(fs_hbm.at[idx_s[j]], rows_s[p], gg[p]).wait()
      pltpu.make_async_copy(fd_hbm.at[idx_d[j]], rows_d[p], gg[p]).wait()

    def issue_scatter(j, p):
      pltpu.async_copy(contrib[p], acc.at[idx_d[j]], ss[p], add=True)

    def wait_scatter(j, p):
      pltpu.make_async_copy(contrib[p], acc.at[idx_d[j]], ss[p]).wait()

    if n_heads > 1:
      cvec = [cmask_v[q, pl.ds(0, 16)] for q in range(nq)]
      didx = didx_v[pl.ds(0, 16)]

      def make_edge_body(p):
        def edge_body(i, _):
          e0 = 2 * i
          e1 = e0 + 1
          den_a = zeros16
          den_b = zeros16
          for q in range(nq):
            sa = rows_s[p][e0, pl.ds(q * 16, 16)]
            sb = rows_s[p][e1, pl.ds(q * 16, 16)]
            da = rows_d[p][e0, pl.ds(q * 16, 16)]
            db = rows_d[p][e1, pl.ds(q * 16, 16)]
            ta = sa + da
            tb = sb + db
            la = jnp.maximum(ta, 0.2 * ta)
            lb = jnp.maximum(tb, 0.2 * tb)
            ma = la * attn_q[q]
            mb = lb * attn_q[q]
            for pidx in perms[:3]:
              ma = bfly(ma, pidx)
              mb = bfly(mb, pidx)
            pa = jnp.exp(ma)
            pb = jnp.exp(mb)
            contrib[p][e0, pl.ds(q * 16, 16)] = pa * sa
            contrib[p][e1, pl.ds(q * 16, 16)] = pb * sb
            den_a = den_a + pa * cvec[q]
            den_b = den_b + pb * cvec[q]
          contrib[p][e0, pl.ds(width, 16)] = jnp.take_along_axis(
              den_a, didx, axis=0, mode="promise_in_bounds")
          contrib[p][e1, pl.ds(width, 16)] = jnp.take_along_axis(
              den_b, didx, axis=0, mode="promise_in_bounds")
          return 0
        return edge_body
      eloop_n, eloop_unroll = CHUNK // 2, 4
    else:
      def make_edge_body(p):
        def edge_body(i, _):
          e0 = 2 * i
          e1 = e0 + 1
          sa_qs = []
          sb_qs = []
          tot_a = zeros16
          tot_b = zeros16
          for q in range(nq):
            sa = rows_s[p][e0, pl.ds(q * 16, 16)]
            sb = rows_s[p][e1, pl.ds(q * 16, 16)]
            da = rows_d[p][e0, pl.ds(q * 16, 16)]
            db = rows_d[p][e1, pl.ds(q * 16, 16)]
            ta = sa + da
            tb = sb + db
            la = jnp.maximum(ta, 0.2 * ta)
            lb = jnp.maximum(tb, 0.2 * tb)
            tot_a = tot_a + la * attn_q[q]
            tot_b = tot_b + lb * attn_q[q]
            sa_qs.append(sa)
            sb_qs.append(sb)
          for p_idx in perms:
            tot_a = bfly(tot_a, p_idx)
            tot_b = bfly(tot_b, p_idx)
          pa = jnp.exp(tot_a)
          pb = jnp.exp(tot_b)
          for q in range(nq):
            contrib[p][e0, pl.ds(q * 16, 16)] = pa * sa_qs[q]
            contrib[p][e1, pl.ds(q * 16, 16)] = pb * sb_qs[q]
          contrib[p][e0, pl.ds(width, 16)] = pa
          contrib[p][e1, pl.ds(width, 16)] = pb
          return 0
        return edge_body
      eloop_n, eloop_unroll = CHUNK // 2, 4

    edge_bodies = [make_edge_body(0), make_edge_body(1)]

    def sub_iter(c, k, steady):
      p = k % 2
      q = 1 - p
      kn = (k + 1) % 4
      kf = (k + 2) % 4
      wait_idx(c + 1, kn)
      issue_gather(kn, q)
      wait_gather(k, p)
      if steady:
        wait_scatter(kf, p)
      lax.fori_loop(0, eloop_n, edge_bodies[p], 0, unroll=eloop_unroll)
      issue_scatter(k, p)
      issue_idx(c + 2, kf)

    issue_idx(0, 0)
    wait_idx(0, 0)
    issue_gather(0, 0)
    issue_idx(1, 1)

    sub_iter(0, 0, False)
    sub_iter(1, 1, False)
    sub_iter(2, 2, True)
    sub_iter(3, 3, True)

    def group_body(g, _):
      c0 = 4 * g
      sub_iter(c0, 0, True)
      sub_iter(c0 + 1, 1, True)
      sub_iter(c0 + 2, 2, True)
      sub_iter(c0 + 3, 3, True)
      return 0
    lax.fori_loop(1, NCHUNK // 4, group_body, 0)

    wait_gather(0, 0)
    wait_idx(NCHUNK + 1, 1)
    wait_scatter(2, 0)
    wait_scatter(3, 1)

    plsc.subcore_barrier()
    for r in range(RFULL):
      off = sid * RPT + r * CHUNK
      pltpu.sync_copy(acc.at[pl.ds(off, CHUNK)],
                      out_hbm.at[cid, pl.ds(off, CHUNK)])
    off = sid * RPT + RFULL * CHUNK
    pltpu.sync_copy(acc.at[pl.ds(off, RTAIL)],
                    out_hbm.at[cid, pl.ds(off, RTAIL)])

  return edge_kernel


def _mm1_body(x_ref, ws_ref, wd_ref, fs_ref, fd_ref):
  xb = x_ref[...]
  fs_ref[...] = jnp.dot(xb, ws_ref[...], preferred_element_type=jnp.float32)
  fd_ref[...] = jnp.dot(xb, wd_ref[...], preferred_element_type=jnp.float32)


def _mid_body(acc_ref, b1_ref, ws_ref, wd_ref, fs2_ref, fd2_ref):
  a = acc_ref[...]
  s = a[0] + a[1]
  num = s[:, :64]
  den = jnp.repeat(s[:, 64:72], 8, axis=1)
  nz = den != 0.0
  h = jnp.where(nz, num / jnp.where(nz, den, 1.0), 0.0) + b1_ref[...]
  h = jnp.where(h > 0.0, h, jnp.exp(h) - 1.0)
  fs2_ref[...] = jnp.dot(h, ws_ref[...], preferred_element_type=jnp.float32)
  fd2_ref[...] = jnp.dot(h, wd_ref[...], preferred_element_type=jnp.float32)


def _fin_body(acc_ref, b2_ref, o_ref):
  a = acc_ref[...]
  s = a[0] + a[1]
  num = s[:, :48]
  den = s[:, 48:49]
  nz = den != 0.0
  o_ref[...] = jnp.where(nz, num / jnp.where(nz, den, 1.0), 0.0) + b2_ref[...]


def kernel(x, edge_index, W1_src, W1_dst, attn1, b1, W2_src, W2_dst,
           attn2, b2):
  f32 = jnp.float32
  npad = NP - N
  x_p = jnp.pad(x, ((0, npad), (0, 0)))
  pad_e = jnp.full((EP - E,), N, jnp.int32)
  src = jnp.concatenate([edge_index[0], pad_e])
  dst = jnp.concatenate([edge_index[1], pad_e])

  cmask_l = [[0.0] * 16 for _ in range(4)]
  for q in range(4):
    cmask_l[q][2 * q] = 1.0
    cmask_l[q][2 * q + 9] = 1.0
  cmask = jnp.array(cmask_l, f32)
  didx = jnp.array([l + 8 if (l % 2 == 1 and l < 8) else l
                    for l in range(16)], jnp.int32)

  attn1_flat = attn1.reshape(H1 * F1).astype(f32)
  attn2_flat = jnp.pad(attn2.reshape(H2 * F2), (0, 8)).astype(f32)
  W2s_p = jnp.pad(W2_src, ((0, 0), (0, 8)))
  W2d_p = jnp.pad(W2_dst, ((0, 0), (0, 8)))
  b1_2d = b1.reshape(1, 64)
  b2_2d = jnp.pad(b2, (0, 8)).reshape(1, 48)

  grid = (NP // BLK,)
  fs1, fd1 = pl.pallas_call(
      _mm1_body,
      grid=grid,
      in_specs=[
          pl.BlockSpec((BLK, D_IN), lambda i: (i, 0)),
          pl.BlockSpec((D_IN, 64), lambda i: (0, 0)),
          pl.BlockSpec((D_IN, 64), lambda i: (0, 0)),
      ],
      out_specs=[
          pl.BlockSpec((BLK, 64), lambda i: (i, 0)),
          pl.BlockSpec((BLK, 64), lambda i: (i, 0)),
      ],
      out_shape=[jax.ShapeDtypeStruct((NP, 64), f32)] * 2,
  )(x_p, W1_src, W1_dst)

  edge1 = _make_sc_edge_kernel(64, H1)
  acc1 = edge1(src, dst, fs1, fd1, attn1_flat, cmask, didx)

  agrid = (ACCN // ABLK,)
  fs2, fd2 = pl.pallas_call(
      _mid_body,
      grid=agrid,
      in_specs=[
          pl.BlockSpec((2, ABLK, 80), lambda i: (0, i, 0)),
          pl.BlockSpec((1, 64), lambda i: (0, 0)),
          pl.BlockSpec((64, 48), lambda i: (0, 0)),
          pl.BlockSpec((64, 48), lambda i: (0, 0)),
      ],
      out_specs=[
          pl.BlockSpec((ABLK, 48), lambda i: (i, 0)),
          pl.BlockSpec((ABLK, 48), lambda i: (i, 0)),
      ],
      out_shape=[jax.ShapeDtypeStruct((ACCN, 48), f32)] * 2,
  )(acc1, b1_2d, W2s_p, W2d_p)

  edge2 = _make_sc_edge_kernel(48, H2)
  acc2 = edge2(src, dst, fs2, fd2, attn2_flat, cmask, didx)

  out = pl.pallas_call(
      _fin_body,
      grid=agrid,
      in_specs=[
          pl.BlockSpec((2, ABLK, 64), lambda i: (0, i, 0)),
          pl.BlockSpec((1, 48), lambda i: (0, 0)),
      ],
      out_specs=pl.BlockSpec((ABLK, 48), lambda i: (i, 0)),
      out_shape=jax.ShapeDtypeStruct((ACCN, 48), f32),
  )(acc2, b2_2d)

  return out[:N, :H2 * F2]


# speedup vs baseline: 2.0532x; 1.3149x over previous
import functools

import jax
import jax.numpy as jnp
from jax import lax
from jax.experimental import pallas as pl
from jax.experimental.pallas import tpu as pltpu
from jax.experimental.pallas import tpu_sc as plsc

N = 10000
E = 320000
D_IN = 128
H1, F1 = 8, 8
H2, F2 = 1, 40

NP = 10240
TILES = 32
CHUNK = 128
ACCN = 10048
RPT = ACCN // 16
RFULL = RPT // CHUNK
RTAIL = RPT - RFULL * CHUNK
ABLK = 1256
NCHUNK = (((E + TILES * CHUNK - 1) // (TILES * CHUNK)) + 3) // 4 * 4
EPT = NCHUNK * CHUNK
EP = EPT * TILES
BLK = 1024


def _make_sc_edge_kernel(width, n_heads):
  accw = width + 16
  nq = width // 16
  mesh = plsc.VectorSubcoreMesh(core_axis_name="c", subcore_axis_name="s",
                                num_cores=2, num_subcores=16)

  @functools.partial(
      pl.kernel,
      out_type=jax.ShapeDtypeStruct((2, ACCN, accw), jnp.float32),
      mesh=mesh,
      compiler_params=pltpu.CompilerParams(use_tc_tiling_on_sc=False),
      scratch_types=[
          pltpu.VMEM((CHUNK,), jnp.int32),
          pltpu.VMEM((CHUNK,), jnp.int32),
          pltpu.VMEM((CHUNK,), jnp.int32),
          pltpu.VMEM((CHUNK,), jnp.int32),
          pltpu.VMEM((CHUNK,), jnp.int32),
          pltpu.VMEM((CHUNK,), jnp.int32),
          pltpu.VMEM((CHUNK,), jnp.int32),
          pltpu.VMEM((CHUNK,), jnp.int32),
          pltpu.VMEM((CHUNK, width), jnp.float32),
          pltpu.VMEM((CHUNK, width), jnp.float32),
          pltpu.VMEM((CHUNK, width), jnp.float32),
          pltpu.VMEM((CHUNK, width), jnp.float32),
          pltpu.VMEM((CHUNK, accw), jnp.float32),
          pltpu.VMEM((CHUNK, accw), jnp.float32),
          pltpu.VMEM((width,), jnp.float32),
          pltpu.VMEM((4, 16), jnp.float32),
          pltpu.VMEM((16,), jnp.int32),
          pltpu.VMEM_SHARED((ACCN, accw), jnp.float32),
          pltpu.SemaphoreType.DMA,
          pltpu.SemaphoreType.DMA,
          pltpu.SemaphoreType.DMA,
          pltpu.SemaphoreType.DMA,
          pltpu.SemaphoreType.DMA,
          pltpu.SemaphoreType.DMA,
          pltpu.SemaphoreType.DMA,
          pltpu.SemaphoreType.DMA,
      ],
  )
  def edge_kernel(src_hbm, dst_hbm, fs_hbm, fd_hbm, attn_hbm,
                  cmask_hbm, didx_hbm, out_hbm,
                  idx_s0, idx_s1, idx_s2, idx_s3,
                  idx_d0, idx_d1, idx_d2, idx_d3,
                  rows_s0, rows_s1, rows_d0, rows_d1,
                  contrib0, contrib1, attn_v, cmask_v, didx_v, acc,
                  ii0, ii1, ii2, ii3, gg0, gg1, ss0, ss1):
    cid = lax.axis_index("c")
    sid = lax.axis_index("s")
    wid = sid * 2 + cid
    tile_base = wid * EPT

    idx_s = [idx_s0, idx_s1, idx_s2, idx_s3]
    idx_d = [idx_d0, idx_d1, idx_d2, idx_d3]
    rows_s = [rows_s0, rows_s1]
    rows_d = [rows_d0, rows_d1]
    contrib = [contrib0, contrib1]
    ii = [ii0, ii1, ii2, ii3]
    gg = [gg0, gg1]
    ss = [ss0, ss1]

    zeros16 = jnp.zeros((16,), jnp.float32)

    def zero_row(i, _):
      def zero_col(j, _):
        contrib0[i, pl.ds(j * 16, 16)] = zeros16
        return 0
      return lax.fori_loop(0, accw // 16, zero_col, 0)
    lax.fori_loop(0, CHUNK, zero_row, 0)
    for r in range(RFULL):
      pltpu.sync_copy(contrib0,
                      acc.at[pl.ds(sid * RPT + r * CHUNK, CHUNK)])
    pltpu.sync_copy(contrib0.at[pl.ds(0, RTAIL)],
                    acc.at[pl.ds(sid * RPT + RFULL * CHUNK, RTAIL)])
    plsc.subcore_barrier()

    pltpu.sync_copy(attn_hbm, attn_v)
    attn_q = [attn_v[pl.ds(q * 16, 16)] for q in range(nq)]
    if n_heads > 1:
      pltpu.sync_copy(cmask_hbm, cmask_v)
      pltpu.sync_copy(didx_hbm, didx_v)

    lanes = lax.iota(jnp.int32, 16)
    perms = [lanes ^ 1, lanes ^ 2, lanes ^ 4, lanes ^ 8]

    def bfly(v, p):
      return v + jnp.take_along_axis(v, p, axis=0,
                                     mode="promise_in_bounds")

    def chunk_base(c):
      return tile_base + jnp.minimum(c, NCHUNK - 1) * CHUNK

    def issue_idx(c, j):
      base = chunk_base(c)
      pltpu.async_copy(src_hbm.at[pl.ds(base, CHUNK)], idx_s[j], ii[j])
      pltpu.async_copy(dst_hbm.at[pl.ds(base, CHUNK)], idx_d[j], ii[j])

    def wait_idx(c, j):
      base = chunk_base(c)
      pltpu.make_async_copy(src_hbm.at[pl.ds(base, CHUNK)], idx_s[j],
                            ii[j]).wait()
      pltpu.make_async_copy(dst_hbm.at[pl.ds(base, CHUNK)], idx_d[j],
                            ii[j]).wait()

    def issue_gather(j, p):
      pltpu.async_copy(fs_hbm.at[idx_s[j]], rows_s[p], gg[p])
      pltpu.async_copy(fd_hbm.at[idx_d[j]], rows_d[p], gg[p])

    def wait_gather(j, p):
      pltpu.make_async_copy(fs_hbm.at[idx_s[j]], rows_s[p], gg[p]).wait()
      pltpu.make_async_copy(fd_hbm.at[idx_d[j]], rows_d[p], gg[p]).wait()

    def issue_scatter(j, p):
      pltpu.async_copy(contrib[p], acc.at[idx_d[j]], ss[p], add=True)

    def wait_scatter(j, p):
      pltpu.make_async_copy(contrib[p], acc.at[idx_d[j]], ss[p]).wait()

    if n_heads > 1:
      cvec = [cmask_v[q, pl.ds(0, 16)] for q in range(nq)]
      didx = didx_v[pl.ds(0, 16)]

      NIL = 4

      def make_edge_body(p):
        def edge_body(i, _):
          es = [NIL * i + j for j in range(NIL)]
          dens = [zeros16] * NIL
          for q in range(nq):
            ss = [rows_s[p][e, pl.ds(q * 16, 16)] for e in es]
            ds_ = [rows_d[p][e, pl.ds(q * 16, 16)] for e in es]
            ts = [s + d for s, d in zip(ss, ds_)]
            ls = [jnp.maximum(t, 0.2 * t) for t in ts]
            ms = [l * attn_q[q] for l in ls]
            for pidx in perms[:3]:
              ms = [bfly(m, pidx) for m in ms]
            ps = [jnp.exp(m) for m in ms]
            for j, e in enumerate(es):
              contrib[p][e, pl.ds(q * 16, 16)] = ps[j] * ss[j]
            dens = [den + pv * cvec[q] for den, pv in zip(dens, ps)]
          for j, e in enumerate(es):
            contrib[p][e, pl.ds(width, 16)] = jnp.take_along_axis(
                dens[j], didx, axis=0, mode="promise_in_bounds")
          return 0
        return edge_body
      eloop_n, eloop_unroll = CHUNK // NIL, 2
    else:
      def make_edge_body(p):
        def edge_body(i, _):
          e0 = 2 * i
          e1 = e0 + 1
          sa_qs = []
          sb_qs = []
          tot_a = zeros16
          tot_b = zeros16
          for q in range(nq):
            sa = rows_s[p][e0, pl.ds(q * 16, 16)]
            sb = rows_s[p][e1, pl.ds(q * 16, 16)]
            da = rows_d[p][e0, pl.ds(q * 16, 16)]
            db = rows_d[p][e1, pl.ds(q * 16, 16)]
            ta = sa + da
            tb = sb + db
            la = jnp.maximum(ta, 0.2 * ta)
            lb = jnp.maximum(tb, 0.2 * tb)
            tot_a = tot_a + la * attn_q[q]
            tot_b = tot_b + lb * attn_q[q]
            sa_qs.append(sa)
            sb_qs.append(sb)
          for p_idx in perms:
            tot_a = bfly(tot_a, p_idx)
            tot_b = bfly(tot_b, p_idx)
          pa = jnp.exp(tot_a)
          pb = jnp.exp(tot_b)
          for q in range(nq):
            contrib[p][e0, pl.ds(q * 16, 16)] = pa * sa_qs[q]
            contrib[p][e1, pl.ds(q * 16, 16)] = pb * sb_qs[q]
          contrib[p][e0, pl.ds(width, 16)] = pa
          contrib[p][e1, pl.ds(width, 16)] = pb
          return 0
        return edge_body
      eloop_n, eloop_unroll = CHUNK // 2, 4

    edge_bodies = [make_edge_body(0), make_edge_body(1)]

    def sub_iter(c, k, steady):
      p = k % 2
      q = 1 - p
      kn = (k + 1) % 4
      kf = (k + 2) % 4
      wait_idx(c + 1, kn)
      issue_gather(kn, q)
      wait_gather(k, p)
      if steady:
        wait_scatter(kf, p)
      lax.fori_loop(0, eloop_n, edge_bodies[p], 0, unroll=eloop_unroll)
      issue_scatter(k, p)
      issue_idx(c + 2, kf)

    issue_idx(0, 0)
    wait_idx(0, 0)
    issue_gather(0, 0)
    issue_idx(1, 1)

    sub_iter(0, 0, False)
    sub_iter(1, 1, False)
    sub_iter(2, 2, True)
    sub_iter(3, 3, True)

    def group_body(g, _):
      c0 = 4 * g
      sub_iter(c0, 0, True)
      sub_iter(c0 + 1, 1, True)
      sub_iter(c0 + 2, 2, True)
      sub_iter(c0 + 3, 3, True)
      return 0
    lax.fori_loop(1, NCHUNK // 4, group_body, 0)

    wait_gather(0, 0)
    wait_idx(NCHUNK + 1, 1)
    wait_scatter(2, 0)
    wait_scatter(3, 1)

    plsc.subcore_barrier()
    for r in range(RFULL):
      off = sid * RPT + r * CHUNK
      pltpu.sync_copy(acc.at[pl.ds(off, CHUNK)],
                      out_hbm.at[cid, pl.ds(off, CHUNK)])
    off = sid * RPT + RFULL * CHUNK
    pltpu.sync_copy(acc.at[pl.ds(off, RTAIL)],
                    out_hbm.at[cid, pl.ds(off, RTAIL)])

  return edge_kernel


def _mm1_body(x_ref, ws_ref, wd_ref, fs_ref, fd_ref):
  xb = x_ref[...]
  fs_ref[...] = jnp.dot(xb, ws_ref[...], preferred_element_type=jnp.float32)
  fd_ref[...] = jnp.dot(xb, wd_ref[...], preferred_element_type=jnp.float32)


def _mid_body(acc_ref, b1_ref, ws_ref, wd_ref, fs2_ref, fd2_ref):
  a = acc_ref[...]
  s = a[0] + a[1]
  num = s[:, :64]
  den = jnp.repeat(s[:, 64:72], 8, axis=1)
  nz = den != 0.0
  h = jnp.where(nz, num / jnp.where(nz, den, 1.0), 0.0) + b1_ref[...]
  h = jnp.where(h > 0.0, h, jnp.exp(h) - 1.0)
  fs2_ref[...] = jnp.dot(h, ws_ref[...], preferred_element_type=jnp.float32)
  fd2_ref[...] = jnp.dot(h, wd_ref[...], preferred_element_type=jnp.float32)


def _fin_body(acc_ref, b2_ref, o_ref):
  a = acc_ref[...]
  s = a[0] + a[1]
  num = s[:, :48]
  den = s[:, 48:49]
  nz = den != 0.0
  o_ref[...] = jnp.where(nz, num / jnp.where(nz, den, 1.0), 0.0) + b2_ref[...]


def kernel(x, edge_index, W1_src, W1_dst, attn1, b1, W2_src, W2_dst,
           attn2, b2):
  f32 = jnp.float32
  npad = NP - N
  x_p = jnp.pad(x, ((0, npad), (0, 0)))
  pad_e = jnp.full((EP - E,), N, jnp.int32)
  src = jnp.concatenate([edge_index[0], pad_e])
  dst = jnp.concatenate([edge_index[1], pad_e])

  cmask_l = [[0.0] * 16 for _ in range(4)]
  for q in range(4):
    cmask_l[q][2 * q] = 1.0
    cmask_l[q][2 * q + 9] = 1.0
  cmask = jnp.array(cmask_l, f32)
  didx = jnp.array([l + 8 if (l % 2 == 1 and l < 8) else l
                    for l in range(16)], jnp.int32)

  attn1_flat = attn1.reshape(H1 * F1).astype(f32)
  attn2_flat = jnp.pad(attn2.reshape(H2 * F2), (0, 8)).astype(f32)
  W2s_p = jnp.pad(W2_src, ((0, 0), (0, 8)))
  W2d_p = jnp.pad(W2_dst, ((0, 0), (0, 8)))
  b1_2d = b1.reshape(1, 64)
  b2_2d = jnp.pad(b2, (0, 8)).reshape(1, 48)

  grid = (NP // BLK,)
  fs1, fd1 = pl.pallas_call(
      _mm1_body,
      grid=grid,
      in_specs=[
          pl.BlockSpec((BLK, D_IN), lambda i: (i, 0)),
          pl.BlockSpec((D_IN, 64), lambda i: (0, 0)),
          pl.BlockSpec((D_IN, 64), lambda i: (0, 0)),
      ],
      out_specs=[
          pl.BlockSpec((BLK, 64), lambda i: (i, 0)),
          pl.BlockSpec((BLK, 64), lambda i: (i, 0)),
      ],
      out_shape=[jax.ShapeDtypeStruct((NP, 64), f32)] * 2,
  )(x_p, W1_src, W1_dst)

  edge1 = _make_sc_edge_kernel(64, H1)
  acc1 = edge1(src, dst, fs1, fd1, attn1_flat, cmask, didx)

  agrid = (ACCN // ABLK,)
  fs2, fd2 = pl.pallas_call(
      _mid_body,
      grid=agrid,
      in_specs=[
          pl.BlockSpec((2, ABLK, 80), lambda i: (0, i, 0)),
          pl.BlockSpec((1, 64), lambda i: (0, 0)),
          pl.BlockSpec((64, 48), lambda i: (0, 0)),
          pl.BlockSpec((64, 48), lambda i: (0, 0)),
      ],
      out_specs=[
          pl.BlockSpec((ABLK, 48), lambda i: (i, 0)),
          pl.BlockSpec((ABLK, 48), lambda i: (i, 0)),
      ],
      out_shape=[jax.ShapeDtypeStruct((ACCN, 48), f32)] * 2,
  )(acc1, b1_2d, W2s_p, W2d_p)

  edge2 = _make_sc_edge_kernel(48, H2)
  acc2 = edge2(src, dst, fs2, fd2, attn2_flat, cmask, didx)

  out = pl.pallas_call(
      _fin_body,
      grid=agrid,
      in_specs=[
          pl.BlockSpec((2, ABLK, 64), lambda i: (0, i, 0)),
          pl.BlockSpec((1, 48), lambda i: (0, 0)),
      ],
      out_specs=pl.BlockSpec((ABLK, 48), lambda i: (i, 0)),
      out_shape=jax.ShapeDtypeStruct((ACCN, 48), f32),
  )(acc2, b2_2d)

  return out[:N, :H2 * F2]


# speedup vs baseline: 2.0990x; 1.0223x over previous
import functools

import jax
import jax.numpy as jnp
from jax import lax
from jax.experimental import pallas as pl
from jax.experimental.pallas import tpu as pltpu
from jax.experimental.pallas import tpu_sc as plsc

N = 10000
E = 320000
D_IN = 128
H1, F1 = 8, 8
H2, F2 = 1, 40

NP = 10240
TILES = 32
CHUNK = 128
ACCN = 10048
RPT = ACCN // 16
RFULL = RPT // CHUNK
RTAIL = RPT - RFULL * CHUNK
ABLK = 1256
NCHUNK = (((E + TILES * CHUNK - 1) // (TILES * CHUNK)) + 3) // 4 * 4
EPT = NCHUNK * CHUNK
EP = EPT * TILES
BLK = 1024


def _make_sc_edge_kernel(width, n_heads):
  accw = width + 16
  nq = width // 16
  mesh = plsc.VectorSubcoreMesh(core_axis_name="c", subcore_axis_name="s",
                                num_cores=2, num_subcores=16)

  @functools.partial(
      pl.kernel,
      out_type=jax.ShapeDtypeStruct((2, ACCN, accw), jnp.float32),
      mesh=mesh,
      compiler_params=pltpu.CompilerParams(use_tc_tiling_on_sc=False),
      scratch_types=[
          pltpu.VMEM((CHUNK,), jnp.int32),
          pltpu.VMEM((CHUNK,), jnp.int32),
          pltpu.VMEM((CHUNK,), jnp.int32),
          pltpu.VMEM((CHUNK,), jnp.int32),
          pltpu.VMEM((CHUNK,), jnp.int32),
          pltpu.VMEM((CHUNK,), jnp.int32),
          pltpu.VMEM((CHUNK,), jnp.int32),
          pltpu.VMEM((CHUNK,), jnp.int32),
          pltpu.VMEM((CHUNK, width), jnp.float32),
          pltpu.VMEM((CHUNK, width), jnp.float32),
          pltpu.VMEM((CHUNK, width), jnp.float32),
          pltpu.VMEM((CHUNK, width), jnp.float32),
          pltpu.VMEM((CHUNK, accw), jnp.float32),
          pltpu.VMEM((CHUNK, accw), jnp.float32),
          pltpu.VMEM((width,), jnp.float32),
          pltpu.VMEM((4, 16), jnp.float32),
          pltpu.VMEM((16,), jnp.int32),
          pltpu.VMEM_SHARED((ACCN, accw), jnp.float32),
          pltpu.SemaphoreType.DMA,
          pltpu.SemaphoreType.DMA,
          pltpu.SemaphoreType.DMA,
          pltpu.SemaphoreType.DMA,
          pltpu.SemaphoreType.DMA,
          pltpu.SemaphoreType.DMA,
          pltpu.SemaphoreType.DMA,
          pltpu.SemaphoreType.DMA,
      ],
  )
  def edge_kernel(src_hbm, dst_hbm, fs_hbm, fd_hbm, attn_hbm,
                  cmask_hbm, didx_hbm, out_hbm,
                  idx_s0, idx_s1, idx_s2, idx_s3,
                  idx_d0, idx_d1, idx_d2, idx_d3,
                  rows_s0, rows_s1, rows_d0, rows_d1,
                  contrib0, contrib1, attn_v, cmask_v, didx_v, acc,
                  ii0, ii1, ii2, ii3, gg0, gg1, ss0, ss1):
    cid = lax.axis_index("c")
    sid = lax.axis_index("s")
    wid = sid * 2 + cid
    tile_base = wid * EPT

    idx_s = [idx_s0, idx_s1, idx_s2, idx_s3]
    idx_d = [idx_d0, idx_d1, idx_d2, idx_d3]
    rows_s = [rows_s0, rows_s1]
    rows_d = [rows_d0, rows_d1]
    contrib = [contrib0, contrib1]
    ii = [ii0, ii1, ii2, ii3]
    gg = [gg0, gg1]
    ss = [ss0, ss1]

    zeros16 = jnp.zeros((16,), jnp.float32)

    def zero_row(i, _):
      def zero_col(j, _):
        contrib0[i, pl.ds(j * 16, 16)] = zeros16
        return 0
      return lax.fori_loop(0, accw // 16, zero_col, 0)
    lax.fori_loop(0, CHUNK, zero_row, 0)
    for r in range(RFULL):
      pltpu.sync_copy(contrib0,
                      acc.at[pl.ds(sid * RPT + r * CHUNK, CHUNK)])
    pltpu.sync_copy(contrib0.at[pl.ds(0, RTAIL)],
                    acc.at[pl.ds(sid * RPT + RFULL * CHUNK, RTAIL)])
    plsc.subcore_barrier()

    pltpu.sync_copy(attn_hbm, attn_v)
    attn_q = [attn_v[pl.ds(q * 16, 16)] for q in range(nq)]
    if n_heads > 1:
      pltpu.sync_copy(cmask_hbm, cmask_v)
      pltpu.sync_copy(didx_hbm, didx_v)

    lanes = lax.iota(jnp.int32, 16)
    perms = [lanes ^ 1, lanes ^ 2, lanes ^ 4, lanes ^ 8]

    def bfly(v, p):
      return v + jnp.take_along_axis(v, p, axis=0,
                                     mode="promise_in_bounds")

    def chunk_base(c):
      return tile_base + jnp.minimum(c, NCHUNK - 1) * CHUNK

    def issue_idx(c, j):
      base = chunk_base(c)
      pltpu.async_copy(src_hbm.at[pl.ds(base, CHUNK)], idx_s[j], ii[j])
      pltpu.async_copy(dst_hbm.at[pl.ds(base, CHUNK)], idx_d[j], ii[j])

    def wait_idx(c, j):
      base = chunk_base(c)
      pltpu.make_async_copy(src_hbm.at[pl.ds(base, CHUNK)], idx_s[j],
                            ii[j]).wait()
      pltpu.make_async_copy(dst_hbm.at[pl.ds(base, CHUNK)], idx_d[j],
                            ii[j]).wait()

    def issue_gather(j, p):
      pltpu.async_copy(fs_hbm.at[idx_s[j]], rows_s[p], gg[p])
      pltpu.async_copy(fd_hbm.at[idx_d[j]], rows_d[p], gg[p])

    def wait_gather(j, p):
      pltpu.make_async_copy(fs_hbm.at[idx_s[j]], rows_s[p], gg[p]).wait()
      pltpu.make_async_copy(fd_hbm.at[idx_d[j]], rows_d[p], gg[p]).wait()

    def issue_scatter(j, p):
      pltpu.async_copy(contrib[p], acc.at[idx_d[j]], ss[p], add=True)

    def wait_scatter(j, p):
      pltpu.make_async_copy(contrib[p], acc.at[idx_d[j]], ss[p]).wait()

    if n_heads > 1:
      cvec = [cmask_v[q, pl.ds(0, 16)] for q in range(nq)]
      didx = didx_v[pl.ds(0, 16)]

      NIL = 4

      def make_edge_body(p):
        def edge_body(i, _):
          es = [NIL * i + j for j in range(NIL)]
          dens = [zeros16] * NIL
          for q in range(nq):
            ss = [rows_s[p][e, pl.ds(q * 16, 16)] for e in es]
            ds_ = [rows_d[p][e, pl.ds(q * 16, 16)] for e in es]
            ts = [s + d for s, d in zip(ss, ds_)]
            ls = [jnp.maximum(t, 0.2 * t) for t in ts]
            ms = [l * attn_q[q] for l in ls]
            for pidx in perms[:3]:
              ms = [bfly(m, pidx) for m in ms]
            ps = [jnp.exp(m) for m in ms]
            for j, e in enumerate(es):
              contrib[p][e, pl.ds(q * 16, 16)] = ps[j] * ss[j]
            dens = [den + pv * cvec[q] for den, pv in zip(dens, ps)]
          for j, e in enumerate(es):
            contrib[p][e, pl.ds(width, 16)] = jnp.take_along_axis(
                dens[j], didx, axis=0, mode="promise_in_bounds")
          return 0
        return edge_body
      eloop_n, eloop_unroll = CHUNK // NIL, 2
    else:
      NIL = 4

      def make_edge_body(p):
        def edge_body(i, _):
          es = [NIL * i + j for j in range(NIL)]
          s_qs = []
          tots = [zeros16] * NIL
          for q in range(nq):
            ss = [rows_s[p][e, pl.ds(q * 16, 16)] for e in es]
            ds_ = [rows_d[p][e, pl.ds(q * 16, 16)] for e in es]
            ts = [s + d for s, d in zip(ss, ds_)]
            ls = [jnp.maximum(t, 0.2 * t) for t in ts]
            tots = [tot + l * attn_q[q] for tot, l in zip(tots, ls)]
            s_qs.append(ss)
          for p_idx in perms:
            tots = [bfly(tot, p_idx) for tot in tots]
          ps = [jnp.exp(tot) for tot in tots]
          for q in range(nq):
            for j, e in enumerate(es):
              contrib[p][e, pl.ds(q * 16, 16)] = ps[j] * s_qs[q][j]
          for j, e in enumerate(es):
            contrib[p][e, pl.ds(width, 16)] = ps[j]
          return 0
        return edge_body
      eloop_n, eloop_unroll = CHUNK // NIL, 2

    edge_bodies = [make_edge_body(0), make_edge_body(1)]

    def sub_iter(c, k, steady):
      p = k % 2
      q = 1 - p
      kn = (k + 1) % 4
      kf = (k + 2) % 4
      wait_idx(c + 1, kn)
      issue_gather(kn, q)
      wait_gather(k, p)
      if steady:
        wait_scatter(kf, p)
      lax.fori_loop(0, eloop_n, edge_bodies[p], 0, unroll=eloop_unroll)
      issue_scatter(k, p)
      issue_idx(c + 2, kf)

    issue_idx(0, 0)
    wait_idx(0, 0)
    issue_gather(0, 0)
    issue_idx(1, 1)

    sub_iter(0, 0, False)
    sub_iter(1, 1, False)
    sub_iter(2, 2, True)
    sub_iter(3, 3, True)

    def group_body(g, _):
      c0 = 4 * g
      sub_iter(c0, 0, True)
      sub_iter(c0 + 1, 1, True)
      sub_iter(c0 + 2, 2, True)
      sub_iter(c0 + 3, 3, True)
      return 0
    lax.fori_loop(1, NCHUNK // 4, group_body, 0)

    wait_gather(0, 0)
    wait_idx(NCHUNK + 1, 1)
    wait_scatter(2, 0)
    wait_scatter(3, 1)

    plsc.subcore_barrier()
    for r in range(RFULL):
      off = sid * RPT + r * CHUNK
      pltpu.sync_copy(acc.at[pl.ds(off, CHUNK)],
                      out_hbm.at[cid, pl.ds(off, CHUNK)])
    off = sid * RPT + RFULL * CHUNK
    pltpu.sync_copy(acc.at[pl.ds(off, RTAIL)],
                    out_hbm.at[cid, pl.ds(off, RTAIL)])

  return edge_kernel


def _mm1_body(x_ref, ws_ref, wd_ref, fs_ref, fd_ref):
  xb = x_ref[...]
  fs_ref[...] = jnp.dot(xb, ws_ref[...], preferred_element_type=jnp.float32)
  fd_ref[...] = jnp.dot(xb, wd_ref[...], preferred_element_type=jnp.float32)


def _mid_body(acc_ref, b1_ref, ws_ref, wd_ref, fs2_ref, fd2_ref):
  a = acc_ref[...]
  s = a[0] + a[1]
  num = s[:, :64]
  den = jnp.repeat(s[:, 64:72], 8, axis=1)
  nz = den != 0.0
  h = jnp.where(nz, num / jnp.where(nz, den, 1.0), 0.0) + b1_ref[...]
  h = jnp.where(h > 0.0, h, jnp.exp(h) - 1.0)
  fs2_ref[...] = jnp.dot(h, ws_ref[...], preferred_element_type=jnp.float32)
  fd2_ref[...] = jnp.dot(h, wd_ref[...], preferred_element_type=jnp.float32)


def _fin_body(acc_ref, b2_ref, o_ref):
  a = acc_ref[...]
  s = a[0] + a[1]
  num = s[:, :48]
  den = s[:, 48:49]
  nz = den != 0.0
  o_ref[...] = jnp.where(nz, num / jnp.where(nz, den, 1.0), 0.0) + b2_ref[...]


def kernel(x, edge_index, W1_src, W1_dst, attn1, b1, W2_src, W2_dst,
           attn2, b2):
  f32 = jnp.float32
  npad = NP - N
  x_p = jnp.pad(x, ((0, npad), (0, 0)))
  pad_e = jnp.full((EP - E,), N, jnp.int32)
  src = jnp.concatenate([edge_index[0], pad_e])
  dst = jnp.concatenate([edge_index[1], pad_e])

  cmask_l = [[0.0] * 16 for _ in range(4)]
  for q in range(4):
    cmask_l[q][2 * q] = 1.0
    cmask_l[q][2 * q + 9] = 1.0
  cmask = jnp.array(cmask_l, f32)
  didx = jnp.array([l + 8 if (l % 2 == 1 and l < 8) else l
                    for l in range(16)], jnp.int32)

  attn1_flat = attn1.reshape(H1 * F1).astype(f32)
  attn2_flat = jnp.pad(attn2.reshape(H2 * F2), (0, 8)).astype(f32)
  W2s_p = jnp.pad(W2_src, ((0, 0), (0, 8)))
  W2d_p = jnp.pad(W2_dst, ((0, 0), (0, 8)))
  b1_2d = b1.reshape(1, 64)
  b2_2d = jnp.pad(b2, (0, 8)).reshape(1, 48)

  grid = (NP // BLK,)
  fs1, fd1 = pl.pallas_call(
      _mm1_body,
      grid=grid,
      in_specs=[
          pl.BlockSpec((BLK, D_IN), lambda i: (i, 0)),
          pl.BlockSpec((D_IN, 64), lambda i: (0, 0)),
          pl.BlockSpec((D_IN, 64), lambda i: (0, 0)),
      ],
      out_specs=[
          pl.BlockSpec((BLK, 64), lambda i: (i, 0)),
          pl.BlockSpec((BLK, 64), lambda i: (i, 0)),
      ],
      out_shape=[jax.ShapeDtypeStruct((NP, 64), f32)] * 2,
  )(x_p, W1_src, W1_dst)

  edge1 = _make_sc_edge_kernel(64, H1)
  acc1 = edge1(src, dst, fs1, fd1, attn1_flat, cmask, didx)

  agrid = (ACCN // ABLK,)
  fs2, fd2 = pl.pallas_call(
      _mid_body,
      grid=agrid,
      in_specs=[
          pl.BlockSpec((2, ABLK, 80), lambda i: (0, i, 0)),
          pl.BlockSpec((1, 64), lambda i: (0, 0)),
          pl.BlockSpec((64, 48), lambda i: (0, 0)),
          pl.BlockSpec((64, 48), lambda i: (0, 0)),
      ],
      out_specs=[
          pl.BlockSpec((ABLK, 48), lambda i: (i, 0)),
          pl.BlockSpec((ABLK, 48), lambda i: (i, 0)),
      ],
      out_shape=[jax.ShapeDtypeStruct((ACCN, 48), f32)] * 2,
  )(acc1, b1_2d, W2s_p, W2d_p)

  edge2 = _make_sc_edge_kernel(48, H2)
  acc2 = edge2(src, dst, fs2, fd2, attn2_flat, cmask, didx)

  out = pl.pallas_call(
      _fin_body,
      grid=agrid,
      in_specs=[
          pl.BlockSpec((2, ABLK, 64), lambda i: (0, i, 0)),
          pl.BlockSpec((1, 48), lambda i: (0, 0)),
      ],
      out_specs=pl.BlockSpec((ABLK, 48), lambda i: (i, 0)),
      out_shape=jax.ShapeDtypeStruct((ACCN, 48), f32),
  )(acc2, b2_2d)

  return out[:N, :H2 * F2]
